# Initial kernel scaffold; baseline (speedup 1.0000x reference)
import jax
import jax.numpy as jnp
from jax.experimental import pallas as pl


def kernel(in1, in2):
    raise NotImplementedError("write your pallas kernel here")


# speedup vs baseline: 1.1377x; 1.1377x over previous
import functools

import jax
import jax.numpy as jnp
from jax import lax
from jax.experimental import pallas as pl
from jax.experimental.pallas import tpu as pltpu
from jax.experimental.pallas import tpu_sc as plsc

N_ROWS = 1600000
NW = 32
ROWS_W = N_ROWS // NW
CHUNK = 2000
N_CHUNKS = ROWS_W // CHUNK

C1 = 0.5773500204086304
C2 = 0.7071099877357483
C3 = 0.40825000405311584
C4 = 0.8165000081062317


def _tp_chunk_compute(a_v, b_v, o_v):
    iota = lax.iota(jnp.int32, 16)
    iota4 = iota * 4
    iota16 = iota * 16

    def grp(gi, carry):
        ibase = gi * 64
        obase = gi * 256
        a0 = plsc.load_gather(a_v, [iota4 + ibase])
        a1 = plsc.load_gather(a_v, [iota4 + (ibase + 1)])
        a2 = plsc.load_gather(a_v, [iota4 + (ibase + 2)])
        a3 = plsc.load_gather(a_v, [iota4 + (ibase + 3)])
        b0 = plsc.load_gather(b_v, [iota4 + ibase])
        b1 = plsc.load_gather(b_v, [iota4 + (ibase + 1)])
        b2 = plsc.load_gather(b_v, [iota4 + (ibase + 2)])
        b3 = plsc.load_gather(b_v, [iota4 + (ibase + 3)])

        p11 = a1 * b1
        p22 = a2 * b2
        p33 = a3 * b3
        p12 = a1 * b2
        p21 = a2 * b1
        p13 = a1 * b3
        p31 = a3 * b1
        p23 = a2 * b3
        p32 = a3 * b2

        outs = (
            a0 * b0,
            C1 * p11 + C1 * p22 + C1 * p33,
            a0 * b1,
            a0 * b2,
            a0 * b3,
            a1 * b0,
            a2 * b0,
            a3 * b0,
            C2 * p23 - C2 * p32,
            C2 * p31 - C2 * p13,
            C2 * p12 - C2 * p21,
            C2 * p31 + C2 * p13,
            C2 * p12 + C2 * p21,
            C4 * p22 - C3 * p11 - C3 * p33,
            C2 * p32 + C2 * p23,
            C2 * p33 - C2 * p11,
        )
        for o, val in enumerate(outs):
            plsc.store_scatter(o_v, [iota16 + (obase + o)], val)
        return carry

    lax.fori_loop(0, CHUNK // 16, grp, 0, unroll=False)


def _make_sc_kernel():
    mesh = plsc.VectorSubcoreMesh(core_axis_name="c", subcore_axis_name="s")

    @functools.partial(
        pl.kernel,
        mesh=mesh,
        compiler_params=pltpu.CompilerParams(
            needs_layout_passes=False, use_tc_tiling_on_sc=False
        ),
        out_type=jax.ShapeDtypeStruct((N_ROWS * 16,), jnp.float32),
        scratch_types=[
            pltpu.VMEM((CHUNK * 4,), jnp.float32),
            pltpu.VMEM((CHUNK * 4,), jnp.float32),
            pltpu.VMEM((CHUNK * 16,), jnp.float32),
        ],
    )
    def sc_tp(in1_hbm, in2_hbm, out_hbm, a_v, b_v, o_v):
        wid = lax.axis_index("s") * 2 + lax.axis_index("c")
        base_row = wid * ROWS_W

        def chunk_body(ci, carry):
            row0 = base_row + ci * CHUNK
            pltpu.sync_copy(in1_hbm.at[pl.ds(row0 * 4, CHUNK * 4)], a_v)
            pltpu.sync_copy(in2_hbm.at[pl.ds(row0 * 4, CHUNK * 4)], b_v)
            _tp_chunk_compute(a_v, b_v, o_v)
            pltpu.sync_copy(o_v, out_hbm.at[pl.ds(row0 * 16, CHUNK * 16)])
            return carry

        lax.fori_loop(0, N_CHUNKS, chunk_body, 0, unroll=False)

    return sc_tp


_SC_TP = _make_sc_kernel()


@jax.jit
def kernel(in1, in2):
    n = in1.shape[0]
    out_flat = _SC_TP(in1.reshape(n * 4), in2.reshape(n * 4))
    return out_flat.reshape(n, 16)


# speedup vs baseline: 21.2498x; 18.6781x over previous
import functools

import jax
import jax.numpy as jnp
from jax import lax
from jax.experimental import pallas as pl
from jax.experimental.pallas import tpu as pltpu
from jax.experimental.pallas import tpu_sc as plsc

N_ROWS = 1600000
N_TILES = N_ROWS // 128
NW = 32
TJ = 4
CHUNKS_W = 98
COLS_C = TJ * 128

C1 = 0.5773500204086304
C2 = 0.7071099877357483
C3 = 0.40825000405311584
C4 = 0.8165000081062317


def _make_sc_kernel():
    mesh = plsc.VectorSubcoreMesh(core_axis_name="c", subcore_axis_name="s")

    @functools.partial(
        pl.kernel,
        mesh=mesh,
        compiler_params=pltpu.CompilerParams(needs_layout_passes=False),
        out_type=jax.ShapeDtypeStruct((16, N_ROWS), jnp.float32),
        scratch_types=[
            pltpu.VMEM((4, COLS_C), jnp.float32),
            pltpu.VMEM((4, COLS_C), jnp.float32),
            pltpu.VMEM((16, COLS_C), jnp.float32),
        ],
    )
    def sc_tp(in1_hbm, in2_hbm, out_hbm, a_v, b_v, o_v):
        wid = lax.axis_index("s") * 2 + lax.axis_index("c")

        def grp(gi, carry):
            c0 = gi * 16
            a0 = a_v[0, pl.ds(c0, 16)]
            a1 = a_v[1, pl.ds(c0, 16)]
            a2 = a_v[2, pl.ds(c0, 16)]
            a3 = a_v[3, pl.ds(c0, 16)]
            b0 = b_v[0, pl.ds(c0, 16)]
            b1 = b_v[1, pl.ds(c0, 16)]
            b2 = b_v[2, pl.ds(c0, 16)]
            b3 = b_v[3, pl.ds(c0, 16)]

            p11 = a1 * b1
            p22 = a2 * b2
            p33 = a3 * b3
            p12 = a1 * b2
            p21 = a2 * b1
            p13 = a1 * b3
            p31 = a3 * b1
            p23 = a2 * b3
            p32 = a3 * b2

            o_v[0, pl.ds(c0, 16)] = a0 * b0
            o_v[1, pl.ds(c0, 16)] = C1 * p11 + C1 * p22 + C1 * p33
            o_v[2, pl.ds(c0, 16)] = a0 * b1
            o_v[3, pl.ds(c0, 16)] = a0 * b2
            o_v[4, pl.ds(c0, 16)] = a0 * b3
            o_v[5, pl.ds(c0, 16)] = a1 * b0
            o_v[6, pl.ds(c0, 16)] = a2 * b0
            o_v[7, pl.ds(c0, 16)] = a3 * b0
            o_v[8, pl.ds(c0, 16)] = C2 * p23 - C2 * p32
            o_v[9, pl.ds(c0, 16)] = C2 * p31 - C2 * p13
            o_v[10, pl.ds(c0, 16)] = C2 * p12 - C2 * p21
            o_v[11, pl.ds(c0, 16)] = C2 * p31 + C2 * p13
            o_v[12, pl.ds(c0, 16)] = C2 * p12 + C2 * p21
            o_v[13, pl.ds(c0, 16)] = C4 * p22 - C3 * p11 - C3 * p33
            o_v[14, pl.ds(c0, 16)] = C2 * p32 + C2 * p23
            o_v[15, pl.ds(c0, 16)] = C2 * p33 - C2 * p11
            return carry

        def chunk_body(k, carry):
            g = wid * CHUNKS_W + k
            start = jnp.minimum(g * TJ, N_TILES - TJ)
            col0 = start * 128
            pltpu.sync_copy(in1_hbm.at[:, pl.ds(col0, COLS_C)], a_v)
            pltpu.sync_copy(in2_hbm.at[:, pl.ds(col0, COLS_C)], b_v)
            lax.fori_loop(0, TJ * 8, grp, 0, unroll=False)
            pltpu.sync_copy(o_v, out_hbm.at[:, pl.ds(col0, COLS_C)])
            return carry

        lax.fori_loop(0, CHUNKS_W, chunk_body, 0, unroll=False)

    return sc_tp


_SC_TP = _make_sc_kernel()


@jax.jit
def kernel(in1, in2):
    return _SC_TP(in1.T, in2.T).T


# speedup vs baseline: 65.2509x; 3.0707x over previous
import functools

import jax
import jax.numpy as jnp
from jax import lax
from jax.experimental import pallas as pl
from jax.experimental.pallas import tpu as pltpu
from jax.experimental.pallas import tpu_sc as plsc

N_ROWS = 1600000
N_TILES = N_ROWS // 128
NW = 32
TJ = 12
CHUNKS_W = 34
H = CHUNKS_W // 2
COLS_C = TJ * 128

C1 = 0.5773500204086304
C2 = 0.7071099877357483
C3 = 0.40825000405311584
C4 = 0.8165000081062317


def _make_sc_kernel():
    mesh = plsc.VectorSubcoreMesh(core_axis_name="c", subcore_axis_name="s")

    @functools.partial(
        pl.kernel,
        mesh=mesh,
        compiler_params=pltpu.CompilerParams(needs_layout_passes=False),
        out_type=jax.ShapeDtypeStruct((16, N_ROWS), jnp.float32),
        scratch_types=[
            pltpu.VMEM((2, 4, COLS_C), jnp.float32),
            pltpu.VMEM((2, 4, COLS_C), jnp.float32),
            pltpu.VMEM((2, 16, COLS_C), jnp.float32),
            pltpu.SemaphoreType.DMA((2,)),
            pltpu.SemaphoreType.DMA((2,)),
            pltpu.SemaphoreType.DMA((2,)),
        ],
    )
    def sc_tp(in1_hbm, in2_hbm, out_hbm, a_v, b_v, o_v, sa, sb, so):
        wid = lax.axis_index("s") * 2 + lax.axis_index("c")

        def col0_of(c):
            g = wid * CHUNKS_W + c
            return jnp.minimum(g * TJ, N_TILES - TJ) * 128

        def start_in(c, p):
            col0 = col0_of(c)
            pltpu.make_async_copy(
                in1_hbm.at[:, pl.ds(col0, COLS_C)], a_v.at[p], sa.at[p]
            ).start()
            pltpu.make_async_copy(
                in2_hbm.at[:, pl.ds(col0, COLS_C)], b_v.at[p], sb.at[p]
            ).start()

        def wait_in(p):
            pltpu.make_async_copy(
                in1_hbm.at[:, pl.ds(0, COLS_C)], a_v.at[p], sa.at[p]
            ).wait()
            pltpu.make_async_copy(
                in2_hbm.at[:, pl.ds(0, COLS_C)], b_v.at[p], sb.at[p]
            ).wait()

        def start_out(c, p):
            col0 = col0_of(c)
            pltpu.make_async_copy(
                o_v.at[p], out_hbm.at[:, pl.ds(col0, COLS_C)], so.at[p]
            ).start()

        def wait_out(p):
            pltpu.make_async_copy(
                o_v.at[p], out_hbm.at[:, pl.ds(0, COLS_C)], so.at[p]
            ).wait()

        def compute(p):
            ap = a_v.at[p]
            bp = b_v.at[p]
            op = o_v.at[p]

            def grp(gi, carry):
                c0 = gi * 16
                a0 = ap[0, pl.ds(c0, 16)]
                a1 = ap[1, pl.ds(c0, 16)]
                a2 = ap[2, pl.ds(c0, 16)]
                a3 = ap[3, pl.ds(c0, 16)]
                b0 = bp[0, pl.ds(c0, 16)]
                b1 = bp[1, pl.ds(c0, 16)]
                b2 = bp[2, pl.ds(c0, 16)]
                b3 = bp[3, pl.ds(c0, 16)]

                p11 = a1 * b1
                p22 = a2 * b2
                p33 = a3 * b3
                p12 = a1 * b2
                p21 = a2 * b1
                p13 = a1 * b3
                p31 = a3 * b1
                p23 = a2 * b3
                p32 = a3 * b2

                op[0, pl.ds(c0, 16)] = a0 * b0
                op[1, pl.ds(c0, 16)] = C1 * p11 + C1 * p22 + C1 * p33
                op[2, pl.ds(c0, 16)] = a0 * b1
                op[3, pl.ds(c0, 16)] = a0 * b2
                op[4, pl.ds(c0, 16)] = a0 * b3
                op[5, pl.ds(c0, 16)] = a1 * b0
                op[6, pl.ds(c0, 16)] = a2 * b0
                op[7, pl.ds(c0, 16)] = a3 * b0
                op[8, pl.ds(c0, 16)] = C2 * p23 - C2 * p32
                op[9, pl.ds(c0, 16)] = C2 * p31 - C2 * p13
                op[10, pl.ds(c0, 16)] = C2 * p12 - C2 * p21
                op[11, pl.ds(c0, 16)] = C2 * p31 + C2 * p13
                op[12, pl.ds(c0, 16)] = C2 * p12 + C2 * p21
                op[13, pl.ds(c0, 16)] = C4 * p22 - C3 * p11 - C3 * p33
                op[14, pl.ds(c0, 16)] = C2 * p32 + C2 * p23
                op[15, pl.ds(c0, 16)] = C2 * p33 - C2 * p11
                return carry

            lax.fori_loop(0, TJ * 8, grp, 0, unroll=False)

        start_in(0, 0)

        def body(k2, carry):
            start_in(2 * k2 + 1, 1)
            wait_in(0)

            @pl.when(k2 > 0)
            def _():
                wait_out(0)

            compute(0)
            start_out(2 * k2, 0)

            @pl.when(k2 < H - 1)
            def _():
                start_in(2 * k2 + 2, 0)

            wait_in(1)

            @pl.when(k2 > 0)
            def _():
                wait_out(1)

            compute(1)
            start_out(2 * k2 + 1, 1)
            return carry

        lax.fori_loop(0, H, body, 0, unroll=False)
        wait_out(0)
        wait_out(1)

    return sc_tp


_SC_TP = _make_sc_kernel()


@jax.jit
def kernel(in1, in2):
    return _SC_TP(in1.T, in2.T).T


# speedup vs baseline: 67.8421x; 1.0397x over previous
import functools

import jax
import jax.numpy as jnp
from jax import lax
from jax.experimental import pallas as pl
from jax.experimental.pallas import tpu as pltpu
from jax.experimental.pallas import tpu_sc as plsc

N_ROWS = 1600000
N_TILES = N_ROWS // 128
NW = 32
TJ = 14
CHUNKS_W = 28
H = CHUNKS_W // 2
COLS_C = TJ * 128

C1 = 0.5773500204086304
C2 = 0.7071099877357483
C3 = 0.40825000405311584
C4 = 0.8165000081062317


def _make_sc_kernel():
    mesh = plsc.VectorSubcoreMesh(core_axis_name="c", subcore_axis_name="s")

    @functools.partial(
        pl.kernel,
        mesh=mesh,
        compiler_params=pltpu.CompilerParams(needs_layout_passes=False),
        out_type=jax.ShapeDtypeStruct((16, N_ROWS), jnp.float32),
        scratch_types=[
            pltpu.VMEM((2, 4, COLS_C), jnp.float32),
            pltpu.VMEM((2, 4, COLS_C), jnp.float32),
            pltpu.VMEM((2, 16, COLS_C), jnp.float32),
            pltpu.SemaphoreType.DMA((2,)),
            pltpu.SemaphoreType.DMA((2,)),
            pltpu.SemaphoreType.DMA((2,)),
        ],
    )
    def sc_tp(in1_hbm, in2_hbm, out_hbm, a_v, b_v, o_v, sa, sb, so):
        wid = lax.axis_index("s") * 2 + lax.axis_index("c")

        def col0_of(c):
            g = wid * CHUNKS_W + c
            return jnp.minimum(g * TJ, N_TILES - TJ) * 128

        def start_in(c, p):
            col0 = col0_of(c)
            pltpu.make_async_copy(
                in1_hbm.at[:, pl.ds(col0, COLS_C)], a_v.at[p], sa.at[p]
            ).start()
            pltpu.make_async_copy(
                in2_hbm.at[:, pl.ds(col0, COLS_C)], b_v.at[p], sb.at[p]
            ).start()

        def wait_in(p):
            pltpu.make_async_copy(
                in1_hbm.at[:, pl.ds(0, COLS_C)], a_v.at[p], sa.at[p]
            ).wait()
            pltpu.make_async_copy(
                in2_hbm.at[:, pl.ds(0, COLS_C)], b_v.at[p], sb.at[p]
            ).wait()

        def start_out(c, p):
            col0 = col0_of(c)
            pltpu.make_async_copy(
                o_v.at[p], out_hbm.at[:, pl.ds(col0, COLS_C)], so.at[p]
            ).start()

        def wait_out(p):
            pltpu.make_async_copy(
                o_v.at[p], out_hbm.at[:, pl.ds(0, COLS_C)], so.at[p]
            ).wait()

        def compute(p):
            ap = a_v.at[p]
            bp = b_v.at[p]
            op = o_v.at[p]

            def grp(gi, carry):
                c0 = gi * 16
                a0 = ap[0, pl.ds(c0, 16)]
                a1 = ap[1, pl.ds(c0, 16)]
                a2 = ap[2, pl.ds(c0, 16)]
                a3 = ap[3, pl.ds(c0, 16)]
                b0 = bp[0, pl.ds(c0, 16)]
                b1 = bp[1, pl.ds(c0, 16)]
                b2 = bp[2, pl.ds(c0, 16)]
                b3 = bp[3, pl.ds(c0, 16)]

                p11 = a1 * b1
                p22 = a2 * b2
                p33 = a3 * b3
                p12 = a1 * b2
                p21 = a2 * b1
                p13 = a1 * b3
                p31 = a3 * b1
                p23 = a2 * b3
                p32 = a3 * b2

                op[0, pl.ds(c0, 16)] = a0 * b0
                op[1, pl.ds(c0, 16)] = C1 * p11 + C1 * p22 + C1 * p33
                op[2, pl.ds(c0, 16)] = a0 * b1
                op[3, pl.ds(c0, 16)] = a0 * b2
                op[4, pl.ds(c0, 16)] = a0 * b3
                op[5, pl.ds(c0, 16)] = a1 * b0
                op[6, pl.ds(c0, 16)] = a2 * b0
                op[7, pl.ds(c0, 16)] = a3 * b0
                op[8, pl.ds(c0, 16)] = C2 * p23 - C2 * p32
                op[9, pl.ds(c0, 16)] = C2 * p31 - C2 * p13
                op[10, pl.ds(c0, 16)] = C2 * p12 - C2 * p21
                op[11, pl.ds(c0, 16)] = C2 * p31 + C2 * p13
                op[12, pl.ds(c0, 16)] = C2 * p12 + C2 * p21
                op[13, pl.ds(c0, 16)] = C4 * p22 - C3 * p11 - C3 * p33
                op[14, pl.ds(c0, 16)] = C2 * p32 + C2 * p23
                op[15, pl.ds(c0, 16)] = C2 * p33 - C2 * p11
                return carry

            lax.fori_loop(0, TJ * 8, grp, 0, unroll=False)

        start_in(0, 0)

        def body(k2, carry):
            start_in(2 * k2 + 1, 1)
            wait_in(0)

            @pl.when(k2 > 0)
            def _():
                wait_out(0)

            compute(0)
            start_out(2 * k2, 0)

            @pl.when(k2 < H - 1)
            def _():
                start_in(2 * k2 + 2, 0)

            wait_in(1)

            @pl.when(k2 > 0)
            def _():
                wait_out(1)

            compute(1)
            start_out(2 * k2 + 1, 1)
            return carry

        lax.fori_loop(0, H, body, 0, unroll=False)
        wait_out(0)
        wait_out(1)

    return sc_tp


_SC_TP = _make_sc_kernel()


@jax.jit
def kernel(in1, in2):
    return _SC_TP(in1.T, in2.T).T


# speedup vs baseline: 68.1690x; 1.0048x over previous
import functools

import jax
import jax.numpy as jnp
from jax import lax
from jax.experimental import pallas as pl
from jax.experimental.pallas import tpu as pltpu
from jax.experimental.pallas import tpu_sc as plsc

N_ROWS = 1600000
N_TILES = N_ROWS // 128
NW = 32
TJ = 14
CHUNKS_W = 28
H = CHUNKS_W // 2
COLS_C = TJ * 128

C1 = 0.5773500204086304
C2 = 0.7071099877357483
C3 = 0.40825000405311584
C4 = 0.8165000081062317


def _make_sc_kernel():
    mesh = plsc.VectorSubcoreMesh(core_axis_name="c", subcore_axis_name="s")

    @functools.partial(
        pl.kernel,
        mesh=mesh,
        compiler_params=pltpu.CompilerParams(
            needs_layout_passes=False,
            disable_bounds_checks=True,
            disable_semaphore_checks=True,
        ),
        out_type=jax.ShapeDtypeStruct((16, N_ROWS), jnp.float32),
        scratch_types=[
            pltpu.VMEM((2, 4, COLS_C), jnp.float32),
            pltpu.VMEM((2, 4, COLS_C), jnp.float32),
            pltpu.VMEM((2, 16, COLS_C), jnp.float32),
            pltpu.SemaphoreType.DMA((2,)),
            pltpu.SemaphoreType.DMA((2,)),
            pltpu.SemaphoreType.DMA((2,)),
        ],
    )
    def sc_tp(in1_hbm, in2_hbm, out_hbm, a_v, b_v, o_v, sa, sb, so):
        wid = lax.axis_index("s") * 2 + lax.axis_index("c")

        def col0_of(c):
            g = wid * CHUNKS_W + c
            return jnp.minimum(g * TJ, N_TILES - TJ) * 128

        def start_in(c, p):
            col0 = col0_of(c)
            pltpu.make_async_copy(
                in1_hbm.at[:, pl.ds(col0, COLS_C)], a_v.at[p], sa.at[p]
            ).start()
            pltpu.make_async_copy(
                in2_hbm.at[:, pl.ds(col0, COLS_C)], b_v.at[p], sb.at[p]
            ).start()

        def wait_in(p):
            pltpu.make_async_copy(
                in1_hbm.at[:, pl.ds(0, COLS_C)], a_v.at[p], sa.at[p]
            ).wait()
            pltpu.make_async_copy(
                in2_hbm.at[:, pl.ds(0, COLS_C)], b_v.at[p], sb.at[p]
            ).wait()

        def start_out(c, p):
            col0 = col0_of(c)
            pltpu.make_async_copy(
                o_v.at[p], out_hbm.at[:, pl.ds(col0, COLS_C)], so.at[p]
            ).start()

        def wait_out(p):
            pltpu.make_async_copy(
                o_v.at[p], out_hbm.at[:, pl.ds(0, COLS_C)], so.at[p]
            ).wait()

        def compute(p):
            ap = a_v.at[p]
            bp = b_v.at[p]
            op = o_v.at[p]

            def grp(gi, carry):
                c0 = gi * 16
                a0 = ap[0, pl.ds(c0, 16)]
                a1 = ap[1, pl.ds(c0, 16)]
                a2 = ap[2, pl.ds(c0, 16)]
                a3 = ap[3, pl.ds(c0, 16)]
                b0 = bp[0, pl.ds(c0, 16)]
                b1 = bp[1, pl.ds(c0, 16)]
                b2 = bp[2, pl.ds(c0, 16)]
                b3 = bp[3, pl.ds(c0, 16)]

                p11 = a1 * b1
                p22 = a2 * b2
                p33 = a3 * b3
                p12 = a1 * b2
                p21 = a2 * b1
                p13 = a1 * b3
                p31 = a3 * b1
                p23 = a2 * b3
                p32 = a3 * b2

                op[0, pl.ds(c0, 16)] = a0 * b0
                op[1, pl.ds(c0, 16)] = C1 * p11 + C1 * p22 + C1 * p33
                op[2, pl.ds(c0, 16)] = a0 * b1
                op[3, pl.ds(c0, 16)] = a0 * b2
                op[4, pl.ds(c0, 16)] = a0 * b3
                op[5, pl.ds(c0, 16)] = a1 * b0
                op[6, pl.ds(c0, 16)] = a2 * b0
                op[7, pl.ds(c0, 16)] = a3 * b0
                op[8, pl.ds(c0, 16)] = C2 * p23 - C2 * p32
                op[9, pl.ds(c0, 16)] = C2 * p31 - C2 * p13
                op[10, pl.ds(c0, 16)] = C2 * p12 - C2 * p21
                op[11, pl.ds(c0, 16)] = C2 * p31 + C2 * p13
                op[12, pl.ds(c0, 16)] = C2 * p12 + C2 * p21
                op[13, pl.ds(c0, 16)] = C4 * p22 - C3 * p11 - C3 * p33
                op[14, pl.ds(c0, 16)] = C2 * p32 + C2 * p23
                op[15, pl.ds(c0, 16)] = C2 * p33 - C2 * p11
                return carry

            lax.fori_loop(0, TJ * 8, grp, 0, unroll=False)

        start_in(0, 0)

        def body(k2, carry):
            start_in(2 * k2 + 1, 1)
            wait_in(0)

            @pl.when(k2 > 0)
            def _():
                wait_out(0)

            compute(0)
            start_out(2 * k2, 0)

            @pl.when(k2 < H - 1)
            def _():
                start_in(2 * k2 + 2, 0)

            wait_in(1)

            @pl.when(k2 > 0)
            def _():
                wait_out(1)

            compute(1)
            start_out(2 * k2 + 1, 1)
            return carry

        lax.fori_loop(0, H, body, 0, unroll=False)
        wait_out(0)
        wait_out(1)

    return sc_tp


_SC_TP = _make_sc_kernel()


@jax.jit
def kernel(in1, in2):
    return _SC_TP(in1.T, in2.T).T
